# baseline (speedup 1.0000x reference)
import functools

import jax
import jax.numpy as jnp
import numpy as np
from jax import lax
from jax.experimental import pallas as pl
from jax.experimental.pallas import tpu as pltpu
from jax.experimental.pallas import tpu_sc as plsc

_PRIMES = (100003, 100019, 100043, 100049, 100057, 100069, 100103, 100109)
_OFFSETS = tuple(int(x) for x in np.concatenate(
    [np.zeros(1, dtype=np.int64), np.cumsum(np.array(_PRIMES, dtype=np.int64))[:-1]]))
_MULTS = (2654435761, 2246822519, 3266489917)
_NUM_HEADS = 8
_HEAD_DIM = 64
_N_PER_NGRAM = 4
_KERNEL = 4

_NC = 2
_NS = 16
_NW = _NC * _NS
_L = 16


def _u32(x):
    return jnp.uint32(x)


def _lo_hi(idv, mult):
    m = int(mult)
    mh, ml = m >> 16, m & 0xFFFF
    a = idv >> _u32(16)
    b = idv & _u32(0xFFFF)
    x = a * _u32(ml) + b * _u32(mh)
    y = b * _u32(ml)
    hi = a * _u32(mh) + ((x + (y >> _u32(16))) >> _u32(16))
    lo = idv * _u32(m & 0xFFFFFFFF)
    return lo, hi


def _head_index(lo, hi, head):
    p = _PRIMES[head]
    c1 = (1 << 32) % p
    c2 = (1 << 40) % p
    hip = jnp.where(hi >= _u32(p), hi - _u32(p), hi)
    a2 = hip >> _u32(8)
    b2 = hip & _u32(255)
    r1 = lax.rem(a2 * _u32(c2) + b2 * _u32(c1), _u32(p))
    r2 = lax.rem(lo, _u32(p))
    s = r1 + r2
    s = jnp.where(s >= _u32(p), s - _u32(p), s)
    return (s + _u32(_OFFSETS[head])).astype(jnp.int32)


def _sc_hash_gather(s0, s1, s2, table):
    n_tok = s0.shape[0]
    tpw = n_tok // _NW
    n_groups = tpw // _L
    row_w = 2 * _HEAD_DIM
    mesh = plsc.VectorSubcoreMesh(core_axis_name="c", subcore_axis_name="s")

    @functools.partial(
        pl.kernel,
        out_type=jax.ShapeDtypeStruct((n_tok * _NUM_HEADS, row_w),
                                      jnp.float32),
        mesh=mesh,
        compiler_params=pltpu.CompilerParams(needs_layout_passes=False,
                                             use_tc_tiling_on_sc=True),
        scratch_types=[
            pltpu.VMEM((tpw,), jnp.uint32),
            pltpu.VMEM((tpw,), jnp.uint32),
            pltpu.VMEM((tpw,), jnp.uint32),
            pltpu.VMEM((n_groups, _L * _NUM_HEADS), jnp.int32),
            pltpu.VMEM((_L * _NUM_HEADS, row_w), jnp.float32),
            pltpu.VMEM((_L * _NUM_HEADS, row_w), jnp.float32),
            pltpu.SemaphoreType.DMA,
            pltpu.SemaphoreType.DMA,
            pltpu.SemaphoreType.DMA,
            pltpu.SemaphoreType.DMA,
        ],
    )
    def k(s0_hbm, s1_hbm, s2_hbm, table_hbm, out_hbm,
          s0_v, s1_v, s2_v, idx_v, buf0, buf1, g0, g1, o0, o1):
        cid = lax.axis_index("c")
        sid = lax.axis_index("s")
        wid = sid * _NC + cid
        base = wid * tpw
        pltpu.sync_copy(s0_hbm.at[pl.ds(base, tpw)], s0_v)
        pltpu.sync_copy(s1_hbm.at[pl.ds(base, tpw)], s1_v)
        pltpu.sync_copy(s2_hbm.at[pl.ds(base, tpw)], s2_v)

        lane = lax.iota(jnp.int32, _L)
        for i in range(n_groups):
            sl = pl.ds(i * _L, _L)
            lo0, hi0 = _lo_hi(s0_v[sl], _MULTS[0])
            lo1, hi1 = _lo_hi(s1_v[sl], _MULTS[1])
            lo2, hi2 = _lo_hi(s2_v[sl], _MULTS[2])
            lo_bi = lo0 ^ lo1
            hi_bi = hi0 ^ hi1
            lo_tri = lo_bi ^ lo2
            hi_tri = hi_bi ^ hi2
            row = jnp.full((_L,), i, dtype=jnp.int32)
            for h in range(_NUM_HEADS):
                lo, hi = (lo_bi, hi_bi) if h < _N_PER_NGRAM else (lo_tri, hi_tri)
                idx = _head_index(lo, hi, h)
                cols = lane * jnp.int32(_NUM_HEADS) + jnp.int32(h)
                plsc.store_scatter(idx_v, [row, cols], idx)

        bufs = (buf0, buf1)
        gsems = (g0, g1)
        osems = (o0, o1)
        rows_per_chunk = _L * _NUM_HEADS
        out_base = base * _NUM_HEADS

        def start_gather(j):
            return pltpu.async_copy(table_hbm.at[idx_v.at[jnp.int32(j)]],
                                    bufs[j % 2], gsems[j % 2])

        def start_out(j):
            return pltpu.async_copy(
                bufs[j % 2],
                out_hbm.at[pl.ds(out_base + j * rows_per_chunk,
                                 rows_per_chunk)],
                osems[j % 2])

        pending_g = {0: start_gather(0)}
        pending_o = {}
        for j in range(n_groups):
            pending_g.pop(j).wait()
            pending_o[j] = start_out(j)
            if j + 1 < n_groups:
                if j >= 1:
                    pending_o.pop(j - 1).wait()
                pending_g[j + 1] = start_gather(j + 1)
        for j in sorted(pending_o):
            pending_o.pop(j).wait()

    return k(s0, s1, s2, table)


def _z():
    return jnp.int32(0)


def _tc_dense(hidden, emb, wk_bf, wv_bf, params):
    b, t, d = hidden.shape
    tb = 512
    nt = t // tb
    e_dim = emb.shape[-1]
    inv_sqrt_d = float(1.0 / np.sqrt(d))

    def body(hid_ref, emb_ref, wk_ref, wv_ref, par_ref, out_ref, carry_ref):
        tj = pl.program_id(1)
        h = hid_ref[0]
        e = emb_ref[0].astype(jnp.bfloat16)
        kp = jnp.dot(e, wk_ref[...], preferred_element_type=jnp.float32)
        vl = jnp.dot(e, wv_ref[...], preferred_element_type=jnp.float32)
        gh = par_ref[4:5, :]
        gk = par_ref[5:6, :]
        hn = jnp.mean(h * h, axis=-1, keepdims=True) + 1e-6
        kn = jnp.mean(kp * kp, axis=-1, keepdims=True) + 1e-6
        dt = jnp.sum((h * gh) * (kp * gk), axis=-1, keepdims=True)
        g = jax.nn.sigmoid(dt * lax.rsqrt(hn) * lax.rsqrt(kn) * inv_sqrt_d)
        mixed = h + g * vl
        @pl.when(tj == 0)
        def _():
            carry_ref[0:_KERNEL - 1, :] = jnp.zeros((_KERNEL - 1, d),
                                                    jnp.float32)

        prev = carry_ref[0:_KERNEL - 1, :]
        full = jnp.concatenate([prev, mixed], axis=0)
        acc = full[0:tb] * par_ref[0:1, :]
        for kk in range(1, _KERNEL):
            acc = acc + full[kk:kk + tb] * par_ref[kk:kk + 1, :]
        out_ref[0] = acc
        carry_ref[0:_KERNEL - 1, :] = mixed[tb - (_KERNEL - 1):tb, :]

    return pl.pallas_call(
        body,
        grid=(b, nt),
        in_specs=[
            pl.BlockSpec((1, tb, d), lambda bi, ti: (bi, ti, _z())),
            pl.BlockSpec((1, tb, e_dim), lambda bi, ti: (bi, ti, _z())),
            pl.BlockSpec((e_dim, d), lambda bi, ti: (_z(), _z())),
            pl.BlockSpec((e_dim, d), lambda bi, ti: (_z(), _z())),
            pl.BlockSpec((8, d), lambda bi, ti: (_z(), _z())),
        ],
        out_specs=pl.BlockSpec((1, tb, d), lambda bi, ti: (bi, ti, _z())),
        out_shape=jax.ShapeDtypeStruct((b, t, d), jnp.float32),
        scratch_shapes=[pltpu.VMEM((8, d), jnp.float32)],
    )(hidden, emb, wk_bf, wv_bf, params)


def _pad_w(w):
    wb = w.astype(jnp.bfloat16).reshape(_NUM_HEADS, _HEAD_DIM, -1)
    wb = jnp.pad(wb, ((0, 0), (0, _HEAD_DIM), (0, 0)))
    return wb.reshape(_NUM_HEADS * 2 * _HEAD_DIM, -1)


def kernel(hidden_states, input_ids, table, W_k, W_v, conv_w, gamma_h,
           gamma_k):
    b, t = input_ids.shape
    d = hidden_states.shape[-1]
    ids = input_ids.astype(jnp.uint32)
    s1 = jnp.pad(ids[:, :-1], ((0, 0), (1, 0)))
    s2 = jnp.pad(ids[:, :-2], ((0, 0), (2, 0)))
    tab128 = jnp.pad(table.astype(jnp.float32), ((0, 0), (0, _HEAD_DIM)))
    emb_rows = _sc_hash_gather(ids.reshape(-1), s1.reshape(-1),
                               s2.reshape(-1), tab128)
    emb = emb_rows.reshape(b, t, _NUM_HEADS * 2 * _HEAD_DIM)
    wk_p = _pad_w(W_k)
    wv_p = _pad_w(W_v)
    params = jnp.concatenate(
        [conv_w.T.astype(jnp.float32),
         gamma_h[None].astype(jnp.float32),
         gamma_k[None].astype(jnp.float32),
         jnp.zeros((2, d), jnp.float32)], axis=0)
    out = _tc_dense(hidden_states, emb, wk_p, wv_p, params)
    return out.astype(jnp.float64)


# speedup vs baseline: 1.1519x; 1.1519x over previous
import functools

import jax
import jax.numpy as jnp
import numpy as np
from jax import lax
from jax.experimental import pallas as pl
from jax.experimental.pallas import tpu as pltpu
from jax.experimental.pallas import tpu_sc as plsc

_PRIMES = (100003, 100019, 100043, 100049, 100057, 100069, 100103, 100109)
_OFFSETS = tuple(int(x) for x in np.concatenate(
    [np.zeros(1, dtype=np.int64), np.cumsum(np.array(_PRIMES, dtype=np.int64))[:-1]]))
_MULTS = (2654435761, 2246822519, 3266489917)
_NUM_HEADS = 8
_HEAD_DIM = 64
_N_PER_NGRAM = 4
_KERNEL = 4

_NC = 2
_NS = 16
_NW = _NC * _NS
_L = 16


def _u32(x):
    return jnp.uint32(x)


def _lo_hi(idv, mult):
    m = int(mult)
    mh, ml = m >> 16, m & 0xFFFF
    a = idv >> _u32(16)
    b = idv & _u32(0xFFFF)
    x = a * _u32(ml) + b * _u32(mh)
    y = b * _u32(ml)
    hi = a * _u32(mh) + ((x + (y >> _u32(16))) >> _u32(16))
    lo = idv * _u32(m & 0xFFFFFFFF)
    return lo, hi


def _head_index(lo, hi, head):
    p = _PRIMES[head]
    c1 = (1 << 32) % p
    c2 = (1 << 40) % p
    hip = jnp.where(hi >= _u32(p), hi - _u32(p), hi)
    a2 = hip >> _u32(8)
    b2 = hip & _u32(255)
    r1 = lax.rem(a2 * _u32(c2) + b2 * _u32(c1), _u32(p))
    r2 = lax.rem(lo, _u32(p))
    s = r1 + r2
    s = jnp.where(s >= _u32(p), s - _u32(p), s)
    return (s + _u32(_OFFSETS[head])).astype(jnp.int32)


def _sc_hash_gather(s0, s1, s2, table):
    n_tok = s0.shape[0]
    tpw = n_tok // _NW
    n_groups = tpw // _L
    rpc = _L * _NUM_HEADS
    mesh = plsc.VectorSubcoreMesh(core_axis_name="c", subcore_axis_name="s")

    @functools.partial(
        pl.kernel,
        out_type=jax.ShapeDtypeStruct((n_tok * _NUM_HEADS, _HEAD_DIM),
                                      jnp.float32),
        mesh=mesh,
        compiler_params=pltpu.CompilerParams(needs_layout_passes=False,
                                             use_tc_tiling_on_sc=True),
        scratch_types=[
            pltpu.VMEM((tpw,), jnp.uint32),
            pltpu.VMEM((tpw,), jnp.uint32),
            pltpu.VMEM((tpw,), jnp.uint32),
            pltpu.VMEM((n_groups, rpc), jnp.int32),
            pltpu.VMEM((rpc, _HEAD_DIM), jnp.float32),
            pltpu.VMEM((rpc, _HEAD_DIM), jnp.float32),
            pltpu.SemaphoreType.DMA,
            pltpu.SemaphoreType.DMA,
            pltpu.SemaphoreType.DMA,
            pltpu.SemaphoreType.DMA,
        ],
    )
    def k(s0_hbm, s1_hbm, s2_hbm, table_hbm, out_hbm,
          s0_v, s1_v, s2_v, idx_v, buf0, buf1, g0, g1, o0, o1):
        cid = lax.axis_index("c")
        sid = lax.axis_index("s")
        wid = sid * _NC + cid
        base = wid * tpw
        pltpu.sync_copy(s0_hbm.at[pl.ds(base, tpw)], s0_v)
        pltpu.sync_copy(s1_hbm.at[pl.ds(base, tpw)], s1_v)
        pltpu.sync_copy(s2_hbm.at[pl.ds(base, tpw)], s2_v)

        lane = lax.iota(jnp.int32, _L)
        for i in range(n_groups):
            sl = pl.ds(i * _L, _L)
            lo0, hi0 = _lo_hi(s0_v[sl], _MULTS[0])
            lo1, hi1 = _lo_hi(s1_v[sl], _MULTS[1])
            lo2, hi2 = _lo_hi(s2_v[sl], _MULTS[2])
            lo_bi = lo0 ^ lo1
            hi_bi = hi0 ^ hi1
            lo_tri = lo_bi ^ lo2
            hi_tri = hi_bi ^ hi2
            row = jnp.full((_L,), i, dtype=jnp.int32)
            for h in range(_NUM_HEADS):
                lo, hi = (lo_bi, hi_bi) if h < _N_PER_NGRAM else (lo_tri, hi_tri)
                idx = _head_index(lo, hi, h)
                cols = lane * jnp.int32(_NUM_HEADS) + jnp.int32(h)
                plsc.store_scatter(idx_v, [row, cols], idx)

        out_base = base * _NUM_HEADS
        bufs = (buf0, buf1)
        gsems = (g0, g1)
        osems = (o0, o1)

        def out_slice(j):
            return out_hbm.at[pl.ds(out_base + j * rpc, rpc), :]

        def enqueue_chunk(j, p):
            for g in range(rpc // _L):
                v = idx_v[j, pl.ds(g * _L, _L)]
                for e in range(_L):
                    r = g * _L + e
                    pltpu.async_copy(table_hbm.at[pl.ds(v[e], 1), :],
                                     bufs[p].at[pl.ds(r, 1), :], gsems[p])

        def drain_gathers(p):
            pltpu.make_async_copy(table_hbm.at[pl.ds(0, rpc), :],
                                  bufs[p], gsems[p]).wait()

        def wait_outcopy(p):
            pltpu.make_async_copy(bufs[p], out_slice(0), osems[p]).wait()

        def body(_, j0):
            j1 = j0 + jnp.int32(1)

            @pl.when(j0 > 0)
            def _():
                wait_outcopy(0)
                wait_outcopy(1)

            enqueue_chunk(j0, 0)
            enqueue_chunk(j1, 1)
            drain_gathers(0)
            pltpu.async_copy(buf0, out_slice(j0), o0)
            drain_gathers(1)
            pltpu.async_copy(buf1, out_slice(j1), o1)
            return j0 + jnp.int32(2)

        lax.fori_loop(0, n_groups // 2, body, jnp.int32(0))
        wait_outcopy(0)
        wait_outcopy(1)

    return k(s0, s1, s2, table)


def _z():
    return jnp.int32(0)


def _tc_dense(hidden, emb, wk_bf, wv_bf, params):
    b, t, d = hidden.shape
    tb = 512
    nt = t // tb
    e_dim = emb.shape[-1]
    inv_sqrt_d = float(1.0 / np.sqrt(d))

    def body(hid_ref, emb_ref, wk_ref, wv_ref, par_ref, out_ref, carry_ref):
        tj = pl.program_id(1)
        h = hid_ref[0]
        e = emb_ref[0].astype(jnp.bfloat16)
        kp = jnp.dot(e, wk_ref[...], preferred_element_type=jnp.float32)
        vl = jnp.dot(e, wv_ref[...], preferred_element_type=jnp.float32)
        gh = par_ref[4:5, :]
        gk = par_ref[5:6, :]
        hn = jnp.mean(h * h, axis=-1, keepdims=True) + 1e-6
        kn = jnp.mean(kp * kp, axis=-1, keepdims=True) + 1e-6
        dt = jnp.sum((h * gh) * (kp * gk), axis=-1, keepdims=True)
        g = jax.nn.sigmoid(dt * lax.rsqrt(hn) * lax.rsqrt(kn) * inv_sqrt_d)
        mixed = h + g * vl
        @pl.when(tj == 0)
        def _():
            carry_ref[0:_KERNEL - 1, :] = jnp.zeros((_KERNEL - 1, d),
                                                    jnp.float32)

        prev = carry_ref[0:_KERNEL - 1, :]
        full = jnp.concatenate([prev, mixed], axis=0)
        acc = full[0:tb] * par_ref[0:1, :]
        for kk in range(1, _KERNEL):
            acc = acc + full[kk:kk + tb] * par_ref[kk:kk + 1, :]
        out_ref[0] = acc
        carry_ref[0:_KERNEL - 1, :] = mixed[tb - (_KERNEL - 1):tb, :]

    return pl.pallas_call(
        body,
        grid=(b, nt),
        in_specs=[
            pl.BlockSpec((1, tb, d), lambda bi, ti: (bi, ti, _z())),
            pl.BlockSpec((1, tb, e_dim), lambda bi, ti: (bi, ti, _z())),
            pl.BlockSpec((e_dim, d), lambda bi, ti: (_z(), _z())),
            pl.BlockSpec((e_dim, d), lambda bi, ti: (_z(), _z())),
            pl.BlockSpec((8, d), lambda bi, ti: (_z(), _z())),
        ],
        out_specs=pl.BlockSpec((1, tb, d), lambda bi, ti: (bi, ti, _z())),
        out_shape=jax.ShapeDtypeStruct((b, t, d), jnp.float32),
        scratch_shapes=[pltpu.VMEM((8, d), jnp.float32)],
    )(hidden, emb, wk_bf, wv_bf, params)


def kernel(hidden_states, input_ids, table, W_k, W_v, conv_w, gamma_h,
           gamma_k):
    b, t = input_ids.shape
    d = hidden_states.shape[-1]
    ids = input_ids.astype(jnp.uint32)
    s1 = jnp.pad(ids[:, :-1], ((0, 0), (1, 0)))
    s2 = jnp.pad(ids[:, :-2], ((0, 0), (2, 0)))
    emb_rows = _sc_hash_gather(ids.reshape(-1), s1.reshape(-1),
                               s2.reshape(-1), table)
    emb = emb_rows.reshape(b, t, _NUM_HEADS * _HEAD_DIM)
    wk_p = W_k.astype(jnp.bfloat16)
    wv_p = W_v.astype(jnp.bfloat16)
    params = jnp.concatenate(
        [conv_w.T.astype(jnp.float32),
         gamma_h[None].astype(jnp.float32),
         gamma_k[None].astype(jnp.float32),
         jnp.zeros((2, d), jnp.float32)], axis=0)
    out = _tc_dense(hidden_states, emb, wk_p, wv_p, params)
    return out.astype(jnp.float64)


# speedup vs baseline: 1.1560x; 1.0035x over previous
import functools

import jax
import jax.numpy as jnp
import numpy as np
from jax import lax
from jax.experimental import pallas as pl
from jax.experimental.pallas import tpu as pltpu
from jax.experimental.pallas import tpu_sc as plsc

_PRIMES = (100003, 100019, 100043, 100049, 100057, 100069, 100103, 100109)
_OFFSETS = tuple(int(x) for x in np.concatenate(
    [np.zeros(1, dtype=np.int64), np.cumsum(np.array(_PRIMES, dtype=np.int64))[:-1]]))
_MULTS = (2654435761, 2246822519, 3266489917)
_NUM_HEADS = 8
_HEAD_DIM = 64
_N_PER_NGRAM = 4
_KERNEL = 4

_NC = 2
_NS = 16
_NW = _NC * _NS
_L = 16


def _u32(x):
    return jnp.uint32(x)


def _lo_hi(idv, mult):
    m = int(mult)
    mh, ml = m >> 16, m & 0xFFFF
    a = idv >> _u32(16)
    b = idv & _u32(0xFFFF)
    x = a * _u32(ml) + b * _u32(mh)
    y = b * _u32(ml)
    hi = a * _u32(mh) + ((x + (y >> _u32(16))) >> _u32(16))
    lo = idv * _u32(m & 0xFFFFFFFF)
    return lo, hi


def _head_index(lo, hi, head):
    p = _PRIMES[head]
    c1 = (1 << 32) % p
    c2 = (1 << 40) % p
    hip = jnp.where(hi >= _u32(p), hi - _u32(p), hi)
    a2 = hip >> _u32(8)
    b2 = hip & _u32(255)
    r1 = lax.rem(a2 * _u32(c2) + b2 * _u32(c1), _u32(p))
    r2 = lax.rem(lo, _u32(p))
    s = r1 + r2
    s = jnp.where(s >= _u32(p), s - _u32(p), s)
    return (s + _u32(_OFFSETS[head])).astype(jnp.int32)


def _sc_hash_gather(s0, s1, s2, table):
    n_tok = s0.shape[0]
    tpw = n_tok // _NW
    n_groups = tpw // _L
    rpc = _L * _NUM_HEADS
    mesh = plsc.VectorSubcoreMesh(core_axis_name="c", subcore_axis_name="s")

    @functools.partial(
        pl.kernel,
        out_type=jax.ShapeDtypeStruct((n_tok * _NUM_HEADS, _HEAD_DIM),
                                      jnp.float32),
        mesh=mesh,
        compiler_params=pltpu.CompilerParams(needs_layout_passes=False,
                                             use_tc_tiling_on_sc=True),
        scratch_types=[
            pltpu.VMEM((tpw,), jnp.uint32),
            pltpu.VMEM((tpw,), jnp.uint32),
            pltpu.VMEM((tpw,), jnp.uint32),
            pltpu.VMEM((n_groups, rpc), jnp.int32),
            pltpu.VMEM((rpc, _HEAD_DIM), jnp.float32),
            pltpu.VMEM((rpc, _HEAD_DIM), jnp.float32),
            pltpu.SemaphoreType.DMA,
            pltpu.SemaphoreType.DMA,
            pltpu.SemaphoreType.DMA,
            pltpu.SemaphoreType.DMA,
        ],
    )
    def k(s0_hbm, s1_hbm, s2_hbm, table_hbm, out_hbm,
          s0_v, s1_v, s2_v, idx_v, buf0, buf1, g0, g1, o0, o1):
        cid = lax.axis_index("c")
        sid = lax.axis_index("s")
        wid = sid * _NC + cid
        base = wid * tpw
        pltpu.sync_copy(s0_hbm.at[pl.ds(base, tpw)], s0_v)
        pltpu.sync_copy(s1_hbm.at[pl.ds(base, tpw)], s1_v)
        pltpu.sync_copy(s2_hbm.at[pl.ds(base, tpw)], s2_v)

        lane = lax.iota(jnp.int32, _L)
        for i in range(n_groups):
            sl = pl.ds(i * _L, _L)
            lo0, hi0 = _lo_hi(s0_v[sl], _MULTS[0])
            lo1, hi1 = _lo_hi(s1_v[sl], _MULTS[1])
            lo2, hi2 = _lo_hi(s2_v[sl], _MULTS[2])
            lo_bi = lo0 ^ lo1
            hi_bi = hi0 ^ hi1
            lo_tri = lo_bi ^ lo2
            hi_tri = hi_bi ^ hi2
            row = jnp.full((_L,), i, dtype=jnp.int32)
            for h in range(_NUM_HEADS):
                lo, hi = (lo_bi, hi_bi) if h < _N_PER_NGRAM else (lo_tri, hi_tri)
                idx = _head_index(lo, hi, h)
                cols = lane * jnp.int32(_NUM_HEADS) + jnp.int32(h)
                plsc.store_scatter(idx_v, [row, cols], idx)

        out_base = base * _NUM_HEADS
        bufs = (buf0, buf1)
        gsems = (g0, g1)
        osems = (o0, o1)

        def out_slice(j):
            return out_hbm.at[pl.ds(out_base + j * rpc, rpc), :]

        def enqueue_chunk(j, p):
            for g in range(rpc // _L):
                v = idx_v[j, pl.ds(g * _L, _L)]
                for e in range(_L):
                    r = g * _L + e
                    pltpu.async_copy(table_hbm.at[pl.ds(v[e], 1), :],
                                     bufs[p].at[pl.ds(r, 1), :], gsems[p])

        def drain_gathers(p):
            pltpu.make_async_copy(table_hbm.at[pl.ds(0, rpc), :],
                                  bufs[p], gsems[p]).wait()

        def wait_outcopy(p):
            pltpu.make_async_copy(bufs[p], out_slice(0), osems[p]).wait()

        def body(_, j0):
            j1 = j0 + jnp.int32(1)

            @pl.when(j0 > 0)
            def _():
                wait_outcopy(0)
                wait_outcopy(1)

            enqueue_chunk(j0, 0)
            enqueue_chunk(j1, 1)
            drain_gathers(0)
            pltpu.async_copy(buf0, out_slice(j0), o0)
            drain_gathers(1)
            pltpu.async_copy(buf1, out_slice(j1), o1)
            return j0 + jnp.int32(2)

        lax.fori_loop(0, n_groups // 2, body, jnp.int32(0))
        wait_outcopy(0)
        wait_outcopy(1)

    return k(s0, s1, s2, table)


def _z():
    return jnp.int32(0)


def _tc_dense(hidden, emb, wk_bf, wv_bf, params):
    b, t, d = hidden.shape
    tb = 512
    nt = t // tb
    e_dim = emb.shape[-1]
    inv_sqrt_d = float(1.0 / np.sqrt(d))

    def body(hid_ref, emb_ref, wk_ref, wv_ref, par_ref, out_ref, carry_ref):
        tj = pl.program_id(1)
        h = hid_ref[0]
        e = emb_ref[0].astype(jnp.bfloat16)
        kp = jnp.dot(e, wk_ref[...], preferred_element_type=jnp.float32)
        vl = jnp.dot(e, wv_ref[...], preferred_element_type=jnp.float32)
        g2 = par_ref[4:5, :]
        hn = jnp.mean(h * h, axis=-1, keepdims=True) + 1e-6
        kn = jnp.mean(kp * kp, axis=-1, keepdims=True) + 1e-6
        dt = jnp.sum((h * g2) * kp, axis=-1, keepdims=True)
        g = jax.nn.sigmoid(dt * lax.rsqrt(hn) * lax.rsqrt(kn) * inv_sqrt_d)
        mixed = h + g * vl
        @pl.when(tj == 0)
        def _():
            carry_ref[0:_KERNEL - 1, :] = jnp.zeros((_KERNEL - 1, d),
                                                    jnp.float32)

        prev = carry_ref[0:_KERNEL - 1, :]
        full = jnp.concatenate([prev, mixed], axis=0)
        acc = full[0:tb] * par_ref[0:1, :]
        for kk in range(1, _KERNEL):
            acc = acc + full[kk:kk + tb] * par_ref[kk:kk + 1, :]
        out_ref[0] = acc
        carry_ref[0:_KERNEL - 1, :] = mixed[tb - (_KERNEL - 1):tb, :]

    return pl.pallas_call(
        body,
        grid=(b, nt),
        in_specs=[
            pl.BlockSpec((1, tb, d), lambda bi, ti: (bi, ti, _z())),
            pl.BlockSpec((1, tb, e_dim), lambda bi, ti: (bi, ti, _z())),
            pl.BlockSpec((e_dim, d), lambda bi, ti: (_z(), _z())),
            pl.BlockSpec((e_dim, d), lambda bi, ti: (_z(), _z())),
            pl.BlockSpec((8, d), lambda bi, ti: (_z(), _z())),
        ],
        out_specs=pl.BlockSpec((1, tb, d), lambda bi, ti: (bi, ti, _z())),
        out_shape=jax.ShapeDtypeStruct((b, t, d), jnp.float32),
        scratch_shapes=[pltpu.VMEM((8, d), jnp.float32)],
    )(hidden, emb, wk_bf, wv_bf, params)


def kernel(hidden_states, input_ids, table, W_k, W_v, conv_w, gamma_h,
           gamma_k):
    b, t = input_ids.shape
    d = hidden_states.shape[-1]
    ids = input_ids.astype(jnp.uint32)
    s1 = jnp.pad(ids[:, :-1], ((0, 0), (1, 0)))
    s2 = jnp.pad(ids[:, :-2], ((0, 0), (2, 0)))
    emb_rows = _sc_hash_gather(ids.reshape(-1), s1.reshape(-1),
                               s2.reshape(-1), table)
    emb = emb_rows.reshape(b, t, _NUM_HEADS * _HEAD_DIM)
    wk_p = W_k.astype(jnp.bfloat16)
    wv_p = W_v.astype(jnp.bfloat16)
    params = jnp.concatenate(
        [conv_w.T.astype(jnp.float32),
         (gamma_h * gamma_k)[None].astype(jnp.float32),
         jnp.zeros((3, d), jnp.float32)], axis=0)
    out = _tc_dense(hidden_states, emb, wk_p, wv_p, params)
    return out.astype(jnp.float64)
